# baseline (speedup 1.0000x reference)
import functools

import jax
import jax.numpy as jnp
from jax.experimental import pallas as pl
from jax.experimental.pallas import tpu as pltpu

_VMEM_LIMIT = 48 * 1024 * 1024


def _round_up(x, m):
    return ((x + m - 1) // m) * m


def _stats_kernel(x_ref, w_ref, sum_ref, sq_ref, *, tp, p_total):
    j = pl.program_id(1)

    @pl.when(j == 0)
    def _init():
        sum_ref[...] = jnp.zeros_like(sum_ref)
        sq_ref[...] = jnp.zeros_like(sq_ref)

    x = x_ref[...]
    if tp > p_total:
        col = jax.lax.broadcasted_iota(jnp.int32, (1, tp), 1)
        x = jnp.where(col < p_total, x, jnp.zeros_like(x))
    u = jnp.dot(w_ref[...].astype(jnp.bfloat16), x.astype(jnp.bfloat16),
                preferred_element_type=jnp.float32)
    sum_ref[...] += jnp.sum(u, axis=1, keepdims=True)
    sq_ref[...] += jnp.sum(u * u, axis=1, keepdims=True)


def _apply_kernel(x_ref, w_ref, sum_ref, sq_ref, g_ref, b_ref, o_ref,
                  scale_ref, shift_ref, *, count, eps):
    j = pl.program_id(1)

    @pl.when(j == 0)
    def _fold():
        tot = sum_ref[0] + sum_ref[1]
        tot2 = sq_ref[0] + sq_ref[1]
        mean = tot * (1.0 / count)
        var = jnp.maximum(tot2 * (1.0 / count) - mean * mean, 0.0)
        scale = g_ref[...] * jax.lax.rsqrt(var + jnp.float32(eps))
        scale_ref[...] = scale
        shift_ref[...] = b_ref[...] - mean * scale

    u = jnp.dot(w_ref[...].astype(jnp.bfloat16),
                x_ref[...].astype(jnp.bfloat16),
                preferred_element_type=jnp.float32)
    z = u * scale_ref[...] + shift_ref[...]
    o_ref[...] = jnp.where(z >= 0, z, 0.01 * z).astype(o_ref.dtype)


def kernel(x_nchw, w_conv, b_conv, gamma, beta, eps=1e-5):
    N, Cin, H, W = x_nchw.shape
    Cout = w_conv.shape[0]
    P = H * W
    del b_conv

    x3 = x_nchw.reshape(N, Cin, P)
    w2 = w_conv.reshape(Cout, Cin)
    g2 = gamma.reshape(Cout, 1)
    b2 = beta.reshape(Cout, 1)

    S = 2 if N % 2 == 0 else 1
    NS = N // S
    tp = _round_up(P, 128)
    count = float(N * P)

    x_idx = lambda s, j: (s * NS + j, 0, 0)
    cparams = pltpu.CompilerParams(
        dimension_semantics=("parallel", "arbitrary"),
        vmem_limit_bytes=_VMEM_LIMIT,
    )

    stats_sum, stats_sq = pl.pallas_call(
        functools.partial(_stats_kernel, tp=tp, p_total=P),
        out_shape=(jax.ShapeDtypeStruct((S, Cout, 1), jnp.float32),
                   jax.ShapeDtypeStruct((S, Cout, 1), jnp.float32)),
        grid=(S, NS),
        in_specs=[pl.BlockSpec((None, Cin, tp), x_idx),
                  pl.BlockSpec((Cout, Cin), lambda s, j: (0, 0))],
        out_specs=(pl.BlockSpec((None, Cout, 1), lambda s, j: (s, 0, 0)),
                   pl.BlockSpec((None, Cout, 1), lambda s, j: (s, 0, 0))),
        compiler_params=cparams,
    )(x3, w2)

    out3 = pl.pallas_call(
        functools.partial(_apply_kernel, count=count, eps=eps),
        out_shape=jax.ShapeDtypeStruct((N, Cout, P), x_nchw.dtype),
        grid=(S, NS),
        in_specs=[pl.BlockSpec((None, Cin, tp), x_idx),
                  pl.BlockSpec((Cout, Cin), lambda s, j: (0, 0)),
                  pl.BlockSpec((S, Cout, 1), lambda s, j: (0, 0, 0)),
                  pl.BlockSpec((S, Cout, 1), lambda s, j: (0, 0, 0)),
                  pl.BlockSpec((Cout, 1), lambda s, j: (0, 0)),
                  pl.BlockSpec((Cout, 1), lambda s, j: (0, 0))],
        out_specs=pl.BlockSpec((None, Cout, tp), x_idx),
        scratch_shapes=[pltpu.VMEM((Cout, 1), jnp.float32),
                        pltpu.VMEM((Cout, 1), jnp.float32)],
        compiler_params=cparams,
    )(x3, w2, stats_sum, stats_sq, g2, b2)

    return out3.reshape(N, Cout, H, W)


# speedup vs baseline: 1.0713x; 1.0713x over previous
import functools

import jax
import jax.numpy as jnp
from jax.experimental import pallas as pl
from jax.experimental.pallas import tpu as pltpu

_VMEM_LIMIT = 56 * 1024 * 1024


def _fused_kernel(x_ref, w_ref, g_ref, b_ref, o_ref,
                  xb_ref, ssum_ref, ssq_ref, scale_ref, shift_ref,
                  *, n, count, eps):
    j = pl.program_id(0)
    wb = w_ref[...].astype(jnp.bfloat16)

    @pl.when(j == 0)
    def _init():
        ssum_ref[...] = jnp.zeros_like(ssum_ref)
        ssq_ref[...] = jnp.zeros_like(ssq_ref)

    @pl.when(j < n)
    def _ingest():
        xb = x_ref[...].astype(jnp.bfloat16)
        xb_ref[pl.ds(j, 1)] = xb[None]
        u = jnp.dot(wb, xb, preferred_element_type=jnp.float32)
        ssum_ref[...] += jnp.sum(u, axis=1, keepdims=True)
        ssq_ref[...] += jnp.sum(u * u, axis=1, keepdims=True)

    @pl.when(j == n)
    def _fold():
        mean = ssum_ref[...] * (1.0 / count)
        var = jnp.maximum(ssq_ref[...] * (1.0 / count) - mean * mean, 0.0)
        scale = g_ref[...] * jax.lax.rsqrt(var + jnp.float32(eps))
        scale_ref[...] = scale
        shift_ref[...] = b_ref[...] - mean * scale

    @pl.when(j >= n)
    def _emit():
        xb = xb_ref[j - n]
        u = jnp.dot(wb, xb, preferred_element_type=jnp.float32)
        z = u * scale_ref[...] + shift_ref[...]
        o_ref[...] = jnp.where(z >= 0, z, 0.01 * z).astype(o_ref.dtype)


def kernel(x_nchw, w_conv, b_conv, gamma, beta, eps=1e-5):
    N, Cin, H, W = x_nchw.shape
    Cout = w_conv.shape[0]
    P = H * W
    del b_conv

    x3 = x_nchw.reshape(N, Cin, P)
    w2 = w_conv.reshape(Cout, Cin)
    g2 = gamma.reshape(Cout, 1)
    b2 = beta.reshape(Cout, 1)
    count = float(N * P)

    out3 = pl.pallas_call(
        functools.partial(_fused_kernel, n=N, count=count, eps=eps),
        out_shape=jax.ShapeDtypeStruct((N, Cout, P), x_nchw.dtype),
        grid=(2 * N,),
        in_specs=[
            pl.BlockSpec((None, Cin, P), lambda j: (jnp.minimum(j, N - 1), 0, 0)),
            pl.BlockSpec((Cout, Cin), lambda j: (0, 0)),
            pl.BlockSpec((Cout, 1), lambda j: (0, 0)),
            pl.BlockSpec((Cout, 1), lambda j: (0, 0)),
        ],
        out_specs=pl.BlockSpec((None, Cout, P),
                               lambda j: (jnp.maximum(j - N, 0), 0, 0)),
        scratch_shapes=[
            pltpu.VMEM((N, Cin, P), jnp.bfloat16),
            pltpu.VMEM((Cout, 1), jnp.float32),
            pltpu.VMEM((Cout, 1), jnp.float32),
            pltpu.VMEM((Cout, 1), jnp.float32),
            pltpu.VMEM((Cout, 1), jnp.float32),
        ],
        compiler_params=pltpu.CompilerParams(
            dimension_semantics=("arbitrary",),
            vmem_limit_bytes=_VMEM_LIMIT,
        ),
    )(x3, w2, g2, b2)

    return out3.reshape(N, Cout, H, W)


# speedup vs baseline: 1.1352x; 1.0596x over previous
import functools

import jax
import jax.numpy as jnp
from jax.experimental import pallas as pl
from jax.experimental.pallas import tpu as pltpu

_VMEM_LIMIT = 56 * 1024 * 1024


def _fused_kernel(x_ref, w_ref, g_ref, b_ref, o_ref,
                  xb_ref, ssum_ref, ssq_ref, scale_ref, shift_ref,
                  *, bs, nsteps, count, eps):
    j = pl.program_id(0)
    wb = w_ref[...].astype(jnp.bfloat16)

    @pl.when(j == 0)
    def _init():
        ssum_ref[...] = jnp.zeros_like(ssum_ref)
        ssq_ref[...] = jnp.zeros_like(ssq_ref)

    @pl.when(j < nsteps)
    def _ingest():
        for s in range(bs):
            xb = x_ref[s].astype(jnp.bfloat16)
            xb_ref[pl.ds(j * bs + s, 1)] = xb[None]
            u = jnp.dot(wb, xb, preferred_element_type=jnp.float32)
            ssum_ref[...] += jnp.sum(u, axis=1, keepdims=True)
            ssq_ref[...] += jnp.sum(u * u, axis=1, keepdims=True)

    @pl.when(j == nsteps)
    def _fold():
        mean = ssum_ref[...] * (1.0 / count)
        var = jnp.maximum(ssq_ref[...] * (1.0 / count) - mean * mean, 0.0)
        scale = g_ref[...] * jax.lax.rsqrt(var + jnp.float32(eps))
        scale_ref[...] = scale
        shift_ref[...] = b_ref[...] - mean * scale

    @pl.when(j >= nsteps)
    def _emit():
        for s in range(bs):
            xb = xb_ref[(j - nsteps) * bs + s]
            u = jnp.dot(wb, xb, preferred_element_type=jnp.float32)
            z = u * scale_ref[...] + shift_ref[...]
            o_ref[s] = jnp.where(z >= 0, z, 0.01 * z).astype(o_ref.dtype)


def kernel(x_nchw, w_conv, b_conv, gamma, beta, eps=1e-5):
    N, Cin, H, W = x_nchw.shape
    Cout = w_conv.shape[0]
    P = H * W
    del b_conv

    x3 = x_nchw.reshape(N, Cin, P)
    w2 = w_conv.reshape(Cout, Cin)
    g2 = gamma.reshape(Cout, 1)
    b2 = beta.reshape(Cout, 1)
    count = float(N * P)

    bs = 2
    nsteps = N // bs

    out3 = pl.pallas_call(
        functools.partial(_fused_kernel, bs=bs, nsteps=nsteps, count=count,
                          eps=eps),
        out_shape=jax.ShapeDtypeStruct((N, Cout, P), x_nchw.dtype),
        grid=(2 * nsteps,),
        in_specs=[
            pl.BlockSpec((bs, Cin, P),
                         lambda j: (jnp.minimum(j, nsteps - 1), 0, 0)),
            pl.BlockSpec((Cout, Cin), lambda j: (0, 0)),
            pl.BlockSpec((Cout, 1), lambda j: (0, 0)),
            pl.BlockSpec((Cout, 1), lambda j: (0, 0)),
        ],
        out_specs=pl.BlockSpec((bs, Cout, P),
                               lambda j: (jnp.maximum(j - nsteps, 0), 0, 0)),
        scratch_shapes=[
            pltpu.VMEM((N, Cin, P), jnp.bfloat16),
            pltpu.VMEM((Cout, 1), jnp.float32),
            pltpu.VMEM((Cout, 1), jnp.float32),
            pltpu.VMEM((Cout, 1), jnp.float32),
            pltpu.VMEM((Cout, 1), jnp.float32),
        ],
        compiler_params=pltpu.CompilerParams(
            dimension_semantics=("arbitrary",),
            vmem_limit_bytes=_VMEM_LIMIT,
        ),
    )(x3, w2, g2, b2)

    return out3.reshape(N, Cout, H, W)
